# Initial kernel scaffold; baseline (speedup 1.0000x reference)
import jax
import jax.numpy as jnp
from jax.experimental import pallas as pl


def kernel(tokens, segment_ids, pos_ids, token_table, segment_table, pos_table):
    raise NotImplementedError("write your pallas kernel here")


# speedup vs baseline: 7.6548x; 7.6548x over previous
import functools

import jax
import jax.numpy as jnp
from jax import lax
from jax.experimental import pallas as pl
from jax.experimental.pallas import tpu as pltpu
from jax.experimental.pallas import tpu_sc as plsc

B = 1024
L = 512
DIM = 128
MAX_LEN = 512
N = B * L

NC = 2
NS = 16
NW = NC * NS
LANES = 16
NSLICE = DIM // LANES

TOK_PER_W = N // NW
CHUNK = 128
NCHUNK = TOK_PER_W // CHUNK

COMB_ROWS = 2 * MAX_LEN
ROWS_PER_W = COMB_ROWS // NW


def _worker_id():
    return lax.axis_index("s") * NC + lax.axis_index("c")


def _combine_body(seg_hbm, pos_hbm, comb_hbm, seg_v, pos_v, out_v):
    w = _worker_id()
    r0 = w * ROWS_PER_W
    s = r0 // MAX_LEN
    p0 = lax.rem(r0, MAX_LEN)
    pltpu.sync_copy(seg_hbm, seg_v)
    pltpu.sync_copy(pos_hbm.at[pl.ds(p0, ROWS_PER_W)], pos_v)

    def row_body(t, carry):
        for j in range(NSLICE):
            sl = pl.ds(j * LANES, LANES)
            out_v[t, sl] = pos_v[t, sl] + seg_v[s, sl]
        return carry

    lax.fori_loop(0, ROWS_PER_W, row_body, 0)
    pltpu.sync_copy(out_v, comb_hbm.at[pl.ds(r0, ROWS_PER_W)])


def _gather_body(tok_hbm, sid_hbm, pid_hbm, table_hbm, comb_hbm, out_hbm,
                 tidx_v, cidx_v, pidx_v, rows_t, rows_c, sem_t, sem_c):
    w = _worker_id()
    base = w * TOK_PER_W

    def chunk_body(g, carry):
        off = base + g * CHUNK
        pltpu.sync_copy(tok_hbm.at[pl.ds(off, CHUNK)], tidx_v)
        pltpu.sync_copy(sid_hbm.at[pl.ds(off, CHUNK)], cidx_v)
        pltpu.sync_copy(pid_hbm.at[pl.ds(off, CHUNK)], pidx_v)
        for j in range(CHUNK // LANES):
            sl = pl.ds(j * LANES, LANES)
            cidx_v[sl] = cidx_v[sl] * MAX_LEN + pidx_v[sl]
        cp_t = pltpu.async_copy(table_hbm.at[tidx_v], rows_t, sem_t)
        cp_c = pltpu.async_copy(comb_hbm.at[cidx_v], rows_c, sem_c)
        cp_t.wait()
        cp_c.wait()

        def tok_body(t, inner):
            for j in range(NSLICE):
                sl = pl.ds(j * LANES, LANES)
                rows_t[t, sl] = rows_t[t, sl] + rows_c[t, sl]
            return inner

        lax.fori_loop(0, CHUNK, tok_body, 0)
        pltpu.sync_copy(rows_t, out_hbm.at[pl.ds(off, CHUNK)])
        return carry

    lax.fori_loop(0, NCHUNK, chunk_body, 0)


def _mesh():
    return plsc.VectorSubcoreMesh(core_axis_name="c", subcore_axis_name="s",
                                  num_cores=NC, num_subcores=NS)


@jax.jit
def kernel(tokens, segment_ids, pos_ids, token_table, segment_table, pos_table):
    tok = tokens.reshape(N).astype(jnp.int32)
    sid = segment_ids.reshape(N).astype(jnp.int32)
    pid = pos_ids.reshape(N).astype(jnp.int32)

    comb = pl.kernel(
        _combine_body,
        out_type=jax.ShapeDtypeStruct((COMB_ROWS, DIM), jnp.float32),
        mesh=_mesh(),
        scratch_types=[
            pltpu.VMEM((2, DIM), jnp.float32),
            pltpu.VMEM((ROWS_PER_W, DIM), jnp.float32),
            pltpu.VMEM((ROWS_PER_W, DIM), jnp.float32),
        ],
    )(segment_table, pos_table)

    out = pl.kernel(
        _gather_body,
        out_type=jax.ShapeDtypeStruct((N, DIM), jnp.float32),
        mesh=_mesh(),
        scratch_types=[
            pltpu.VMEM((CHUNK,), jnp.int32),
            pltpu.VMEM((CHUNK,), jnp.int32),
            pltpu.VMEM((CHUNK,), jnp.int32),
            pltpu.VMEM((CHUNK, DIM), jnp.float32),
            pltpu.VMEM((CHUNK, DIM), jnp.float32),
            pltpu.SemaphoreType.DMA,
            pltpu.SemaphoreType.DMA,
        ],
    )(tok, sid, pid, token_table, comb)

    return out.reshape(B, L, DIM)


# speedup vs baseline: 14.2270x; 1.8586x over previous
import jax
import jax.numpy as jnp
from jax import lax
from jax.experimental import pallas as pl
from jax.experimental.pallas import tpu as pltpu
from jax.experimental.pallas import tpu_sc as plsc

B = 1024
L = 512
DIM = 128
MAX_LEN = 512
N = B * L

NC = 2
NS = 16
NW = NC * NS
LANES = 16
NSLICE = DIM // LANES

TOK_PER_W = N // NW
CHUNK = 128
SUPER = 16
SUPERTOK = SUPER * CHUNK
NSUPER = TOK_PER_W // SUPERTOK

COMB_ROWS = 2 * MAX_LEN
ROWS_PER_W = COMB_ROWS // NW


def _worker_id():
    return lax.axis_index("s") * NC + lax.axis_index("c")


def _combine_body(seg_hbm, pos_hbm, comb_hbm, seg_v, pos_v, out_v):
    w = _worker_id()
    r0 = w * ROWS_PER_W
    s = r0 // MAX_LEN
    p0 = lax.rem(r0, MAX_LEN)
    pltpu.sync_copy(seg_hbm, seg_v)
    pltpu.sync_copy(pos_hbm.at[pl.ds(p0, ROWS_PER_W)], pos_v)

    def row_body(t, carry):
        for j in range(NSLICE):
            sl = pl.ds(j * LANES, LANES)
            out_v[t, sl] = pos_v[t, sl] + seg_v[s, sl]
        return carry

    lax.fori_loop(0, ROWS_PER_W, row_body, 0)
    pltpu.sync_copy(out_v, comb_hbm.at[pl.ds(r0, ROWS_PER_W)])


def _gather_body(tok_hbm, sid_hbm, pid_hbm, table_hbm, comb_hbm, out_hbm,
                 tidx_v, cidx_v, pidx_v,
                 rows_t0, rows_t1, rows_c0, rows_c1, out_v0, out_v1,
                 sem_t0, sem_t1, sem_c0, sem_c1, sem_w0, sem_w1):
    w = _worker_id()
    base = w * TOK_PER_W
    rows_t = (rows_t0, rows_t1)
    rows_c = (rows_c0, rows_c1)
    out_v = (out_v0, out_v1)
    sem_t = (sem_t0, sem_t1)
    sem_c = (sem_c0, sem_c1)
    sem_w = (sem_w0, sem_w1)

    def fire(s, sb, b, k):
        tsl = tidx_v.at[sb, pl.ds(k * CHUNK, CHUNK)]
        csl = cidx_v.at[sb, pl.ds(k * CHUNK, CHUNK)]
        pltpu.async_copy(table_hbm.at[tsl], rows_t[b], sem_t[b])
        pltpu.async_copy(comb_hbm.at[csl], rows_c[b], sem_c[b])

    def wait_gathers(sb, b, k):
        tsl = tidx_v.at[sb, pl.ds(k * CHUNK, CHUNK)]
        csl = cidx_v.at[sb, pl.ds(k * CHUNK, CHUNK)]
        pltpu.make_async_copy(table_hbm.at[tsl], rows_t[b], sem_t[b]).wait()
        pltpu.make_async_copy(comb_hbm.at[csl], rows_c[b], sem_c[b]).wait()

    def wait_write(b):
        pltpu.make_async_copy(out_v[b], out_hbm.at[pl.ds(base, CHUNK)],
                              sem_w[b]).wait()

    def add_and_write(s, b, k):
        def tok_body(t, inner):
            for j in range(NSLICE):
                sl = pl.ds(j * LANES, LANES)
                out_v[b][t, sl] = rows_t[b][t, sl] + rows_c[b][t, sl]
            return inner

        lax.fori_loop(0, CHUNK, tok_body, 0)
        off = base + s * SUPERTOK + k * CHUNK
        pltpu.async_copy(out_v[b], out_hbm.at[pl.ds(off, CHUNK)], sem_w[b])

    for s in range(NSUPER):
        sb = s % 2
        off = base + s * SUPERTOK
        pltpu.sync_copy(tok_hbm.at[pl.ds(off, SUPERTOK)], tidx_v.at[sb])
        pltpu.sync_copy(sid_hbm.at[pl.ds(off, SUPERTOK)], cidx_v.at[sb])
        pltpu.sync_copy(pid_hbm.at[pl.ds(off, SUPERTOK)], pidx_v.at[sb])

        def cidx_body(i, carry, sb=sb):
            sl = pl.ds(i * LANES, LANES)
            cidx_v[sb, sl] = cidx_v[sb, sl] * MAX_LEN + pidx_v[sb, sl]
            return carry

        lax.fori_loop(0, SUPERTOK // LANES, cidx_body, 0)

        fire(s, sb, 0, 0)

        def pair_body(g, carry, s=s, sb=sb):
            ka = 2 * g
            kb = 2 * g + 1
            fire(s, sb, 1, kb)
            wait_gathers(sb, 0, ka)
            if s == 0:
                @pl.when(g > 0)
                def _():
                    wait_write(0)
            else:
                wait_write(0)
            add_and_write(s, 0, ka)

            @pl.when(kb + 1 < SUPER)
            def _():
                fire(s, sb, 0, ka + 2)

            wait_gathers(sb, 1, kb)
            if s == 0:
                @pl.when(g > 0)
                def _():
                    wait_write(1)
            else:
                wait_write(1)
            add_and_write(s, 1, kb)
            return carry

        lax.fori_loop(0, SUPER // 2, pair_body, 0)

    wait_write(0)
    wait_write(1)


def _mesh():
    return plsc.VectorSubcoreMesh(core_axis_name="c", subcore_axis_name="s",
                                  num_cores=NC, num_subcores=NS)


@jax.jit
def kernel(tokens, segment_ids, pos_ids, token_table, segment_table, pos_table):
    tok = tokens.reshape(N).astype(jnp.int32)
    sid = segment_ids.reshape(N).astype(jnp.int32)
    pid = pos_ids.reshape(N).astype(jnp.int32)

    comb = pl.kernel(
        _combine_body,
        out_type=jax.ShapeDtypeStruct((COMB_ROWS, DIM), jnp.float32),
        mesh=_mesh(),
        scratch_types=[
            pltpu.VMEM((2, DIM), jnp.float32),
            pltpu.VMEM((ROWS_PER_W, DIM), jnp.float32),
            pltpu.VMEM((ROWS_PER_W, DIM), jnp.float32),
        ],
    )(segment_table, pos_table)

    out = pl.kernel(
        _gather_body,
        out_type=jax.ShapeDtypeStruct((N, DIM), jnp.float32),
        mesh=_mesh(),
        scratch_types=[
            pltpu.VMEM((2, SUPERTOK), jnp.int32),
            pltpu.VMEM((2, SUPERTOK), jnp.int32),
            pltpu.VMEM((2, SUPERTOK), jnp.int32),
            pltpu.VMEM((CHUNK, DIM), jnp.float32),
            pltpu.VMEM((CHUNK, DIM), jnp.float32),
            pltpu.VMEM((CHUNK, DIM), jnp.float32),
            pltpu.VMEM((CHUNK, DIM), jnp.float32),
            pltpu.VMEM((CHUNK, DIM), jnp.float32),
            pltpu.VMEM((CHUNK, DIM), jnp.float32),
            pltpu.SemaphoreType.DMA,
            pltpu.SemaphoreType.DMA,
            pltpu.SemaphoreType.DMA,
            pltpu.SemaphoreType.DMA,
            pltpu.SemaphoreType.DMA,
            pltpu.SemaphoreType.DMA,
        ],
    )(tok, sid, pid, token_table, comb)

    return out.reshape(B, L, DIM)
